# baseline (speedup 1.0000x reference)
import functools

import jax
import jax.numpy as jnp
from jax import lax
from jax.experimental import pallas as pl
from jax.experimental.pallas import tpu as pltpu
from jax.experimental.pallas import tpu_sc as plsc

SEQ = 200
DIM = 64
NWORKERS = 32
GATHER = 100
SUBS_PER_CHUNK = 8
ROWS_PER_CHUNK = GATHER * SUBS_PER_CHUNK


@functools.lru_cache(maxsize=None)
def _build(total_rows):
    rows_per_worker = total_rows // NWORKERS
    n_chunks = rows_per_worker // ROWS_PER_CHUNK
    groups_per_worker = rows_per_worker // GATHER
    mesh = plsc.VectorSubcoreMesh(core_axis_name="c", subcore_axis_name="s")
    info = plsc.get_sparse_core_info()
    nc = info.num_cores

    @functools.partial(
        pl.kernel,
        out_type=jax.ShapeDtypeStruct((total_rows // GATHER, GATHER, DIM),
                                      jnp.float32),
        mesh=mesh,
        scratch_types=[
            pltpu.VMEM((SUBS_PER_CHUNK, GATHER), jnp.int32),
            pltpu.VMEM((SUBS_PER_CHUNK, GATHER, DIM), jnp.float32),
            pltpu.VMEM((SEQ, DIM), jnp.float32),
            pltpu.SemaphoreType.DMA,
        ],
        compiler_params=pltpu.CompilerParams(use_tc_tiling_on_sc=False),
    )
    def emb(idx_hbm, tok_hbm, pos_hbm, out_hbm, idx_v, rows_v, pos_v, sem):
        wid = lax.axis_index("s") * nc + lax.axis_index("c")
        pltpu.sync_copy(pos_hbm, pos_v)

        def chunk_body(ci, carry):
            gbase = wid * groups_per_worker + ci * SUBS_PER_CHUNK
            pltpu.sync_copy(idx_hbm.at[pl.ds(gbase, SUBS_PER_CHUNK)], idx_v)
            copies = []
            for j in range(SUBS_PER_CHUNK):
                copies.append(
                    pltpu.async_copy(tok_hbm.at[idx_v.at[j]], rows_v.at[j],
                                     sem)
                )
            for c in copies:
                c.wait()

            def pos_body(t, c2):
                for g in range(SUBS_PER_CHUNK):
                    p = (g % 2) * GATHER + t
                    for d in range(DIM // 16):
                        sl = pl.ds(d * 16, 16)
                        rows_v[g, t, sl] = rows_v[g, t, sl] + pos_v[p, sl]
                return c2

            lax.fori_loop(0, GATHER, pos_body, 0)
            pltpu.sync_copy(rows_v, out_hbm.at[pl.ds(gbase, SUBS_PER_CHUNK)])
            return carry

        lax.fori_loop(0, n_chunks, chunk_body, 0)

    return emb


def kernel(inputs, token_table, position_table):
    b, s = inputs.shape
    d = token_table.shape[1]
    total = b * s
    idx2d = inputs.astype(jnp.int32).reshape(total // GATHER, GATHER)
    out = _build(total)(idx2d, token_table, position_table)
    return out.reshape(b, s, d)


# speedup vs baseline: 1.0542x; 1.0542x over previous
import functools

import jax
import jax.numpy as jnp
from jax import lax
from jax.experimental import pallas as pl
from jax.experimental.pallas import tpu as pltpu
from jax.experimental.pallas import tpu_sc as plsc

SEQ = 200
DIM = 64
NWORKERS = 32
GATHER = 100
SUBS = 4
ROWS_PER_CHUNK = GATHER * SUBS
NBUF = 4


@functools.lru_cache(maxsize=None)
def _build(total_rows):
    n_chunks_total = total_rows // ROWS_PER_CHUNK
    n_chunks = n_chunks_total // NWORKERS
    mesh = plsc.VectorSubcoreMesh(core_axis_name="c", subcore_axis_name="s")
    info = plsc.get_sparse_core_info()
    nc = info.num_cores

    @functools.partial(
        pl.kernel,
        out_type=jax.ShapeDtypeStruct(
            (n_chunks_total, SUBS, GATHER, DIM), jnp.float32),
        mesh=mesh,
        scratch_types=[
            pltpu.VMEM((NBUF, SUBS, GATHER), jnp.int32),
            pltpu.VMEM((NBUF, SUBS, GATHER, DIM), jnp.float32),
            pltpu.VMEM((SEQ, DIM), jnp.float32),
            [pltpu.SemaphoreType.DMA] * NBUF,
            [pltpu.SemaphoreType.DMA] * NBUF,
        ],
        compiler_params=pltpu.CompilerParams(use_tc_tiling_on_sc=False),
    )
    def emb(idx_hbm, tok_hbm, pos_hbm, out_hbm, idx_v, rows_v, pos_v, sg, sw):
        wid = lax.axis_index("s") * nc + lax.axis_index("c")
        cbase = wid * n_chunks
        pltpu.sync_copy(pos_hbm, pos_v)

        def stage_and_fire(ci, b):
            pltpu.sync_copy(idx_hbm.at[cbase + ci], idx_v.at[b])
            for j in range(SUBS):
                pltpu.async_copy(tok_hbm.at[idx_v.at[b].at[j]],
                                 rows_v.at[b].at[j], sg[b])

        def drain_gathers(b):
            for j in range(SUBS):
                pltpu.make_async_copy(tok_hbm.at[idx_v.at[b].at[j]],
                                      rows_v.at[b].at[j], sg[b]).wait()

        def add_pos(b):
            def pos_body(t, c):
                for g in range(SUBS):
                    p = (g % 2) * GATHER + t
                    for d in range(DIM // 16):
                        sl = pl.ds(d * 16, 16)
                        rows_v[b, g, t, sl] = rows_v[b, g, t, sl] + pos_v[p, sl]
                return c

            lax.fori_loop(0, GATHER, pos_body, 0)

        def fire_wb(ci, b):
            pltpu.async_copy(rows_v.at[b], out_hbm.at[cbase + ci], sw[b])

        def wait_wb(b):
            pltpu.make_async_copy(rows_v.at[b], out_hbm.at[cbase], sw[b]).wait()

        for b in range(NBUF - 1):
            stage_and_fire(b, b)

        def body(k, carry):
            for b in range(NBUF):
                ci = NBUF * k + b
                bp = (b + NBUF - 1) % NBUF

                @pl.when(ci + NBUF - 1 < n_chunks)
                def _prefetch():
                    @pl.when(ci >= 1)
                    def _reclaim():
                        wait_wb(bp)

                    stage_and_fire(ci + NBUF - 1, bp)

                drain_gathers(b)
                add_pos(b)
                fire_wb(ci, b)
            return carry

        lax.fori_loop(0, n_chunks // NBUF, body, 0)
        for b in range(NBUF):
            wait_wb(b)

    return emb


def kernel(inputs, token_table, position_table):
    b, s = inputs.shape
    d = token_table.shape[1]
    total = b * s
    idx3d = inputs.astype(jnp.int32).reshape(
        total // ROWS_PER_CHUNK, SUBS, GATHER)
    out = _build(total)(idx3d, token_table, position_table)
    return out.reshape(b, s, d)
